# baseline (speedup 1.0000x reference)
import functools

import jax
import jax.numpy as jnp
from jax import lax
from jax.experimental import pallas as pl
from jax.experimental.pallas import tpu as pltpu
from jax.experimental.pallas import tpu_sc as plsc

N_AUTHORS = 100000
DIM = 64
N_SLICES = 6
K6 = N_SLICES * DIM
BATCH = 16384

ABLK = 8192
NFULL = N_AUTHORS // ABLK
TAIL = N_AUTHORS - NFULL * ABLK
GRID = NFULL + 1

_info = plsc.get_sparse_core_info()
NC, NS, L = _info.num_cores, _info.num_subcores, _info.num_lanes
NW = NC * NS
P = BATCH // NW
C = 128
NCHUNK = P // C


def _tc_body(fw_ref, feats_any, out_any, x_v, xt_v, y_v, yt_v,
             si0, si1, sti, so0, so1, sto):
    i = pl.program_id(0)
    sis = [si0, si1]
    sos = [so0, so1]

    def in_copy(j, slot):
        return pltpu.make_async_copy(
            feats_any.at[:, pl.ds(j * ABLK, ABLK)], x_v.at[slot], sis[slot])

    def tail_in_copy():
        return pltpu.make_async_copy(
            feats_any.at[:, pl.ds(NFULL * ABLK, TAIL)], xt_v, sti)

    def out_copy(j, slot):
        return pltpu.make_async_copy(
            y_v.at[slot], out_any.at[pl.ds(j * ABLK, ABLK)], sos[slot])

    fwv = fw_ref[...]
    m = jnp.max(fwv, axis=0, keepdims=True)
    e = jnp.exp(fwv - m)
    w = e / jnp.sum(e, axis=0, keepdims=True)
    ws = [w[s // 3, s % 3] for s in range(N_SLICES)]

    def combine(x):
        acc = x[pl.ds(0, DIM), :] * ws[0]
        for s in range(1, N_SLICES):
            acc = acc + x[pl.ds(s * DIM, DIM), :] * ws[s]
        return jnp.transpose(acc, (1, 0))

    @pl.when(i == 0)
    def _prime():
        in_copy(0, 0).start()

    for slot in (0, 1):
        @pl.when(jnp.logical_and(i + 1 < NFULL, (i + 1) % 2 == slot))
        def _prefetch(slot=slot):
            in_copy(i + 1, slot).start()

    @pl.when(i + 1 == NFULL)
    def _prefetch_tail():
        tail_in_copy().start()

    @pl.when(i < NFULL)
    def _full():
        for slot in (0, 1):
            @pl.when(i % 2 == slot)
            def _go(slot=slot):
                in_copy(i, slot).wait()
                y = combine(x_v.at[slot])
                @pl.when(i >= 2)
                def _drain():
                    out_copy(i - 2, slot).wait()
                y_v[slot, :, 0:DIM] = y
                y_v[slot, :, DIM:2 * DIM] = y
                out_copy(i, slot).start()

    @pl.when(i == NFULL)
    def _tail():
        tail_in_copy().wait()
        y = combine(xt_v)
        out_copy(NFULL - 2, 0).wait()
        out_copy(NFULL - 1, 1).wait()
        yt_v[:, 0:DIM] = y
        yt_v[:, DIM:2 * DIM] = y
        pltpu.make_async_copy(
            yt_v, out_any.at[pl.ds(NFULL * ABLK, TAIL)], sto).start()
        pltpu.make_async_copy(
            yt_v, out_any.at[pl.ds(NFULL * ABLK, TAIL)], sto).wait()


def _tc_stage(featsT, fw):
    return pl.pallas_call(
        _tc_body,
        grid=(GRID,),
        in_specs=[
            pl.BlockSpec((2, 3), lambda i: (0, 0)),
            pl.BlockSpec(memory_space=pl.ANY),
        ],
        out_specs=pl.BlockSpec(memory_space=pl.ANY),
        out_shape=jax.ShapeDtypeStruct((N_AUTHORS, 2 * DIM), jnp.float32),
        scratch_shapes=[
            pltpu.VMEM((2, K6, ABLK), jnp.float32),
            pltpu.VMEM((K6, TAIL), jnp.float32),
            pltpu.VMEM((2, ABLK, 2 * DIM), jnp.float32),
            pltpu.VMEM((TAIL, 2 * DIM), jnp.float32),
            pltpu.SemaphoreType.DMA,
            pltpu.SemaphoreType.DMA,
            pltpu.SemaphoreType.DMA,
            pltpu.SemaphoreType.DMA,
            pltpu.SemaphoreType.DMA,
            pltpu.SemaphoreType.DMA,
        ],
    )(fw, featsT)


NSLOT = 3


def _sc_body(emb_hbm, idx_i_hbm, idx_j_hbm, params_hbm, out_hbm,
             par_v, idxi_v, idxj_v, rows_i_v, rows_j_v, dist_v, fold_v,
             sem0, sem1, sem2):
    wid = lax.axis_index("s") * NC + lax.axis_index("c")
    base = wid * P
    pltpu.sync_copy(params_hbm, par_v)
    intercept = par_v[...][0]
    lanes = lax.iota(jnp.int32, 16)
    sems = [sem0, sem1, sem2]

    zero16 = jnp.zeros((L,), jnp.float32)
    for k in range(L):
        fold_v[k, pl.ds(L, L)] = zero16

    for chunk in range(NCHUNK):
        cbase = base + chunk * C
        pltpu.sync_copy(idx_i_hbm.at[pl.ds(cbase, C)], idxi_v.at[chunk])
        pltpu.sync_copy(idx_j_hbm.at[pl.ds(cbase, C)], idxj_v.at[chunk])

    def fire(chunk):
        slot = chunk % NSLOT
        cp_i = pltpu.async_copy(
            emb_hbm.at[idxi_v.at[chunk]], rows_i_v.at[slot], sems[slot])
        cp_j = pltpu.async_copy(
            emb_hbm.at[idxj_v.at[chunk]], rows_j_v.at[slot], sems[slot])
        return cp_i, cp_j

    pending = [fire(c) for c in range(min(NSLOT, NCHUNK))]
    for chunk in range(NCHUNK):
        slot = chunk % NSLOT
        cp_i, cp_j = pending[chunk]
        cp_i.wait()
        cp_j.wait()
        if chunk + NSLOT < NCHUNK:
            pending.append(fire(chunk + NSLOT))

        def group_body(g, _, slot=slot, chunk=chunk):
            dvec = jnp.zeros((L,), jnp.float32)
            for k in range(L):
                c = g * L + k
                sq = None
                for d in range(DIM // L):
                    sl = pl.ds(d * L, L)
                    a = rows_i_v[slot, c, sl] - rows_j_v[slot, c, sl]
                    sq = a * a if sq is None else sq + a * a
                x = sq
                for sh in (8, 4, 2, 1):
                    fold_v[k, pl.ds(0, L)] = x
                    x = x + fold_v[k, pl.ds(sh, L)]
                dvec = dvec + jnp.where(lanes == k, x[0], 0.0)
            dist_v[pl.ds(chunk * C + g * L, L)] = dvec
            return 0

        lax.fori_loop(0, C // L, group_body, 0)

    for k in range(P // L):
        sl = pl.ds(k * L, L)
        d = dist_v[sl]
        dist_v[sl] = 1.0 / (1.0 + jnp.exp(d - intercept))
    pltpu.sync_copy(dist_v, out_hbm.at[pl.ds(base, P)])


def _sc_stage(emb2, idx_i, idx_j, params):
    mesh = plsc.VectorSubcoreMesh(core_axis_name="c", subcore_axis_name="s")
    fn = functools.partial(
        pl.kernel,
        mesh=mesh,
        out_type=jax.ShapeDtypeStruct((BATCH,), jnp.float32),
        scratch_types=[
            pltpu.VMEM((16,), jnp.float32),
            pltpu.VMEM((NCHUNK, C), jnp.int32),
            pltpu.VMEM((NCHUNK, C), jnp.int32),
            pltpu.VMEM((NSLOT, C, 2 * DIM), jnp.float32),
            pltpu.VMEM((NSLOT, C, 2 * DIM), jnp.float32),
            pltpu.VMEM((P,), jnp.float32),
            pltpu.VMEM((L, 2 * L), jnp.float32),
            pltpu.SemaphoreType.DMA,
            pltpu.SemaphoreType.DMA,
            pltpu.SemaphoreType.DMA,
        ],
    )(_sc_body)
    return fn(emb2, idx_i, idx_j, params)


@jax.jit
def kernel(idx_i, idx_j, precomputed_features, feature_weights, intercept):
    featsT = jnp.transpose(
        precomputed_features, (0, 1, 3, 2)).reshape(K6, N_AUTHORS)
    emb2 = _tc_stage(featsT, feature_weights.astype(jnp.float32))
    params = jnp.concatenate([
        intercept.reshape(1).astype(jnp.float32),
        jnp.zeros((15,), jnp.float32),
    ])
    return _sc_stage(emb2, idx_i, idx_j, params)


# speedup vs baseline: 1.0293x; 1.0293x over previous
import functools

import jax
import jax.numpy as jnp
from jax import lax
from jax.experimental import pallas as pl
from jax.experimental.pallas import tpu as pltpu
from jax.experimental.pallas import tpu_sc as plsc

N_AUTHORS = 100000
DIM = 64
N_SLICES = 6
K6 = N_SLICES * DIM
BATCH = 16384

ABLK = 8192
NFULL = N_AUTHORS // ABLK
TAIL = N_AUTHORS - NFULL * ABLK
GRID = NFULL + 1

_info = plsc.get_sparse_core_info()
NC, NS, L = _info.num_cores, _info.num_subcores, _info.num_lanes
NW = NC * NS
P = BATCH // NW
C = 128
NCHUNK = P // C


def _tc_body(fw_ref, feats_any, out_any, x_v, xt_v, y_v, yt_v,
             si0, si1, sti, so0, so1, sto):
    i = pl.program_id(0)
    sis = [si0, si1]
    sos = [so0, so1]

    def in_copy(j, slot):
        return pltpu.make_async_copy(
            feats_any.at[:, pl.ds(j * ABLK, ABLK)], x_v.at[slot], sis[slot])

    def tail_in_copy():
        return pltpu.make_async_copy(
            feats_any.at[:, pl.ds(NFULL * ABLK, TAIL)], xt_v, sti)

    def out_copy(j, slot):
        return pltpu.make_async_copy(
            y_v.at[slot], out_any.at[pl.ds(j * ABLK, ABLK)], sos[slot])

    fwv = fw_ref[...]
    m = jnp.max(fwv, axis=0, keepdims=True)
    e = jnp.exp(fwv - m)
    w = e / jnp.sum(e, axis=0, keepdims=True)
    ws = [w[s // 3, s % 3] for s in range(N_SLICES)]

    def combine(x):
        acc = x[pl.ds(0, DIM), :] * ws[0]
        for s in range(1, N_SLICES):
            acc = acc + x[pl.ds(s * DIM, DIM), :] * ws[s]
        return jnp.transpose(acc, (1, 0))

    @pl.when(i == 0)
    def _prime():
        in_copy(0, 0).start()

    for slot in (0, 1):
        @pl.when(jnp.logical_and(i + 1 < NFULL, (i + 1) % 2 == slot))
        def _prefetch(slot=slot):
            in_copy(i + 1, slot).start()

    @pl.when(i + 1 == NFULL)
    def _prefetch_tail():
        tail_in_copy().start()

    @pl.when(i < NFULL)
    def _full():
        for slot in (0, 1):
            @pl.when(i % 2 == slot)
            def _go(slot=slot):
                in_copy(i, slot).wait()
                y = combine(x_v.at[slot])
                @pl.when(i >= 2)
                def _drain():
                    out_copy(i - 2, slot).wait()
                y_v[slot, :, 0:DIM] = y
                y_v[slot, :, DIM:2 * DIM] = y
                out_copy(i, slot).start()

    @pl.when(i == NFULL)
    def _tail():
        tail_in_copy().wait()
        y = combine(xt_v)
        out_copy(NFULL - 2, 0).wait()
        out_copy(NFULL - 1, 1).wait()
        yt_v[:, 0:DIM] = y
        yt_v[:, DIM:2 * DIM] = y
        pltpu.make_async_copy(
            yt_v, out_any.at[pl.ds(NFULL * ABLK, TAIL)], sto).start()
        pltpu.make_async_copy(
            yt_v, out_any.at[pl.ds(NFULL * ABLK, TAIL)], sto).wait()


def _tc_stage(featsT, fw):
    return pl.pallas_call(
        _tc_body,
        grid=(GRID,),
        in_specs=[
            pl.BlockSpec((2, 3), lambda i: (0, 0)),
            pl.BlockSpec(memory_space=pl.ANY),
        ],
        out_specs=pl.BlockSpec(memory_space=pl.ANY),
        out_shape=jax.ShapeDtypeStruct((N_AUTHORS, 2 * DIM), jnp.float32),
        scratch_shapes=[
            pltpu.VMEM((2, K6, ABLK), jnp.float32),
            pltpu.VMEM((K6, TAIL), jnp.float32),
            pltpu.VMEM((2, ABLK, 2 * DIM), jnp.float32),
            pltpu.VMEM((TAIL, 2 * DIM), jnp.float32),
            pltpu.SemaphoreType.DMA,
            pltpu.SemaphoreType.DMA,
            pltpu.SemaphoreType.DMA,
            pltpu.SemaphoreType.DMA,
            pltpu.SemaphoreType.DMA,
            pltpu.SemaphoreType.DMA,
        ],
    )(fw, featsT)


def _sc_body(emb_hbm, idx_i_hbm, idx_j_hbm, params_hbm, out_hbm,
             par_v, idxi_v, idxj_v, rows_i_v, rows_j_v, dist_v, fold_v,
             sem0, sem1):
    wid = lax.axis_index("s") * NC + lax.axis_index("c")
    base = wid * P
    pltpu.sync_copy(params_hbm, par_v)
    intercept = par_v[...][0]
    lanes = lax.iota(jnp.int32, 16)
    sems = [sem0, sem1]

    zero16 = jnp.zeros((L,), jnp.float32)
    for k in range(L):
        fold_v[k, pl.ds(L, L)] = zero16

    def fire(chunk):
        slot = chunk % 2
        cbase = base + chunk * C
        pltpu.sync_copy(idx_i_hbm.at[pl.ds(cbase, C)], idxi_v.at[slot])
        pltpu.sync_copy(idx_j_hbm.at[pl.ds(cbase, C)], idxj_v.at[slot])
        cp_i = pltpu.async_copy(
            emb_hbm.at[idxi_v.at[slot]], rows_i_v.at[slot], sems[slot])
        cp_j = pltpu.async_copy(
            emb_hbm.at[idxj_v.at[slot]], rows_j_v.at[slot], sems[slot])
        return cp_i, cp_j

    pending = fire(0)
    for chunk in range(NCHUNK):
        slot = chunk % 2
        cp_i, cp_j = pending
        cp_i.wait()
        cp_j.wait()
        if chunk + 1 < NCHUNK:
            pending = fire(chunk + 1)

        def group_body(g, _, slot=slot, chunk=chunk):
            dvec = jnp.zeros((L,), jnp.float32)
            for k in range(L):
                c = g * L + k
                sq = None
                for d in range(DIM // L):
                    sl = pl.ds(d * L, L)
                    a = rows_i_v[slot, c, sl] - rows_j_v[slot, c, sl]
                    sq = a * a if sq is None else sq + a * a
                x = sq
                for sh in (8, 4, 2, 1):
                    fold_v[k, pl.ds(0, L)] = x
                    x = x + fold_v[k, pl.ds(sh, L)]
                dvec = dvec + jnp.where(lanes == k, x[0], 0.0)
            dist_v[pl.ds(chunk * C + g * L, L)] = dvec
            return 0

        lax.fori_loop(0, C // L, group_body, 0)

    for k in range(P // L):
        sl = pl.ds(k * L, L)
        d = dist_v[sl]
        dist_v[sl] = 1.0 / (1.0 + jnp.exp(d - intercept))
    pltpu.sync_copy(dist_v, out_hbm.at[pl.ds(base, P)])


def _sc_stage(emb2, idx_i, idx_j, params):
    mesh = plsc.VectorSubcoreMesh(core_axis_name="c", subcore_axis_name="s")
    fn = functools.partial(
        pl.kernel,
        mesh=mesh,
        out_type=jax.ShapeDtypeStruct((BATCH,), jnp.float32),
        scratch_types=[
            pltpu.VMEM((16,), jnp.float32),
            pltpu.VMEM((2, C), jnp.int32),
            pltpu.VMEM((2, C), jnp.int32),
            pltpu.VMEM((2, C, 2 * DIM), jnp.float32),
            pltpu.VMEM((2, C, 2 * DIM), jnp.float32),
            pltpu.VMEM((P,), jnp.float32),
            pltpu.VMEM((L, 2 * L), jnp.float32),
            pltpu.SemaphoreType.DMA,
            pltpu.SemaphoreType.DMA,
        ],
    )(_sc_body)
    return fn(emb2, idx_i, idx_j, params)


@jax.jit
def kernel(idx_i, idx_j, precomputed_features, feature_weights, intercept):
    featsT = jnp.transpose(
        precomputed_features, (0, 1, 3, 2)).reshape(K6, N_AUTHORS)
    emb2 = _tc_stage(featsT, feature_weights.astype(jnp.float32))
    params = jnp.concatenate([
        intercept.reshape(1).astype(jnp.float32),
        jnp.zeros((15,), jnp.float32),
    ])
    return _sc_stage(emb2, idx_i, idx_j, params)


# speedup vs baseline: 1.0311x; 1.0017x over previous
import functools

import jax
import jax.numpy as jnp
from jax import lax
from jax.experimental import pallas as pl
from jax.experimental.pallas import tpu as pltpu
from jax.experimental.pallas import tpu_sc as plsc

N_AUTHORS = 100000
DIM = 64
N_SLICES = 6
K6 = N_SLICES * DIM
BATCH = 16384

ABLK = 8192
NFULL = N_AUTHORS // ABLK
TAIL = N_AUTHORS - NFULL * ABLK
GRID = NFULL + 1

_info = plsc.get_sparse_core_info()
NC, NS, L = _info.num_cores, _info.num_subcores, _info.num_lanes
NW = NC * NS
P = BATCH // NW
C = 128
NCHUNK = P // C


def _tc_body(fw_ref, feats_any, out_any, x_v, xt_v, y_v, yt_v,
             si0, si1, sti, so0, so1, sto):
    i = pl.program_id(0)
    sis = [si0, si1]
    sos = [so0, so1]

    def in_copy(j, slot):
        return pltpu.make_async_copy(
            feats_any.at[:, pl.ds(j * ABLK, ABLK)], x_v.at[slot], sis[slot])

    def tail_in_copy():
        return pltpu.make_async_copy(
            feats_any.at[:, pl.ds(NFULL * ABLK, TAIL)], xt_v, sti)

    def out_copy(j, slot):
        return pltpu.make_async_copy(
            y_v.at[slot], out_any.at[pl.ds(j * ABLK, ABLK)], sos[slot])

    fwv = fw_ref[...]
    m = jnp.max(fwv, axis=0, keepdims=True)
    e = jnp.exp(fwv - m)
    w = e / jnp.sum(e, axis=0, keepdims=True)
    ws = [w[s // 3, s % 3] for s in range(N_SLICES)]

    def combine(x):
        acc = x[pl.ds(0, DIM), :] * ws[0]
        for s in range(1, N_SLICES):
            acc = acc + x[pl.ds(s * DIM, DIM), :] * ws[s]
        return jnp.transpose(acc, (1, 0))

    @pl.when(i == 0)
    def _prime():
        in_copy(0, 0).start()

    for slot in (0, 1):
        @pl.when(jnp.logical_and(i + 1 < NFULL, (i + 1) % 2 == slot))
        def _prefetch(slot=slot):
            in_copy(i + 1, slot).start()

    @pl.when(i + 1 == NFULL)
    def _prefetch_tail():
        tail_in_copy().start()

    @pl.when(i < NFULL)
    def _full():
        for slot in (0, 1):
            @pl.when(i % 2 == slot)
            def _go(slot=slot):
                in_copy(i, slot).wait()
                y = combine(x_v.at[slot])
                @pl.when(i >= 2)
                def _drain():
                    out_copy(i - 2, slot).wait()
                y_v[slot, :, 0:DIM] = y
                y_v[slot, :, DIM:2 * DIM] = y
                out_copy(i, slot).start()

    @pl.when(i == NFULL)
    def _tail():
        tail_in_copy().wait()
        y = combine(xt_v)
        out_copy(NFULL - 2, 0).wait()
        out_copy(NFULL - 1, 1).wait()
        yt_v[:, 0:DIM] = y
        yt_v[:, DIM:2 * DIM] = y
        pltpu.make_async_copy(
            yt_v, out_any.at[pl.ds(NFULL * ABLK, TAIL)], sto).start()
        pltpu.make_async_copy(
            yt_v, out_any.at[pl.ds(NFULL * ABLK, TAIL)], sto).wait()


def _tc_stage(featsT, fw):
    return pl.pallas_call(
        _tc_body,
        grid=(GRID,),
        in_specs=[
            pl.BlockSpec((2, 3), lambda i: (0, 0)),
            pl.BlockSpec(memory_space=pl.ANY),
        ],
        out_specs=pl.BlockSpec(memory_space=pl.ANY),
        out_shape=jax.ShapeDtypeStruct((N_AUTHORS, 2 * DIM), jnp.float32),
        scratch_shapes=[
            pltpu.VMEM((2, K6, ABLK), jnp.float32),
            pltpu.VMEM((K6, TAIL), jnp.float32),
            pltpu.VMEM((2, ABLK, 2 * DIM), jnp.float32),
            pltpu.VMEM((TAIL, 2 * DIM), jnp.float32),
            pltpu.SemaphoreType.DMA,
            pltpu.SemaphoreType.DMA,
            pltpu.SemaphoreType.DMA,
            pltpu.SemaphoreType.DMA,
            pltpu.SemaphoreType.DMA,
            pltpu.SemaphoreType.DMA,
        ],
    )(fw, featsT)


def _sc_body(emb_hbm, idx_i_hbm, idx_j_hbm, params_hbm, out_hbm,
             par_v, idxi_v, idxj_v, rows_i_v, rows_j_v, dist_v, fold_v,
             sem0, sem1):
    wid = lax.axis_index("s") * NC + lax.axis_index("c")
    base = wid * P
    pltpu.sync_copy(params_hbm, par_v)
    intercept = par_v[...][0]
    lanes = lax.iota(jnp.int32, 16)
    sems = [sem0, sem1]

    zero16 = jnp.zeros((L,), jnp.float32)
    for k in range(L):
        fold_v[k, pl.ds(L, L)] = zero16

    H = C // 2

    def fire(chunk):
        slot = chunk % 2
        cbase = base + chunk * C
        pltpu.sync_copy(idx_i_hbm.at[pl.ds(cbase, C)], idxi_v.at[slot])
        pltpu.sync_copy(idx_j_hbm.at[pl.ds(cbase, C)], idxj_v.at[slot])
        cps = []
        for h in (0, 1):
            hs = pl.ds(h * H, H)
            cps.append(pltpu.async_copy(
                emb_hbm.at[idxi_v.at[slot, hs]],
                rows_i_v.at[slot, hs], sems[slot]))
            cps.append(pltpu.async_copy(
                emb_hbm.at[idxj_v.at[slot, hs]],
                rows_j_v.at[slot, hs], sems[slot]))
        return cps

    pending = fire(0)
    for chunk in range(NCHUNK):
        slot = chunk % 2
        for cp in pending:
            cp.wait()
        if chunk + 1 < NCHUNK:
            pending = fire(chunk + 1)

        def group_body(g, _, slot=slot, chunk=chunk):
            dvec = jnp.zeros((L,), jnp.float32)
            for k in range(L):
                c = g * L + k
                sq = None
                for d in range(DIM // L):
                    sl = pl.ds(d * L, L)
                    a = rows_i_v[slot, c, sl] - rows_j_v[slot, c, sl]
                    sq = a * a if sq is None else sq + a * a
                x = sq
                for sh in (8, 4, 2, 1):
                    fold_v[k, pl.ds(0, L)] = x
                    x = x + fold_v[k, pl.ds(sh, L)]
                dvec = dvec + jnp.where(lanes == k, x[0], 0.0)
            dist_v[pl.ds(chunk * C + g * L, L)] = dvec
            return 0

        lax.fori_loop(0, C // L, group_body, 0)

    for k in range(P // L):
        sl = pl.ds(k * L, L)
        d = dist_v[sl]
        dist_v[sl] = 1.0 / (1.0 + jnp.exp(d - intercept))
    pltpu.sync_copy(dist_v, out_hbm.at[pl.ds(base, P)])


def _sc_stage(emb2, idx_i, idx_j, params):
    mesh = plsc.VectorSubcoreMesh(core_axis_name="c", subcore_axis_name="s")
    fn = functools.partial(
        pl.kernel,
        mesh=mesh,
        out_type=jax.ShapeDtypeStruct((BATCH,), jnp.float32),
        scratch_types=[
            pltpu.VMEM((16,), jnp.float32),
            pltpu.VMEM((2, C), jnp.int32),
            pltpu.VMEM((2, C), jnp.int32),
            pltpu.VMEM((2, C, 2 * DIM), jnp.float32),
            pltpu.VMEM((2, C, 2 * DIM), jnp.float32),
            pltpu.VMEM((P,), jnp.float32),
            pltpu.VMEM((L, 2 * L), jnp.float32),
            pltpu.SemaphoreType.DMA,
            pltpu.SemaphoreType.DMA,
        ],
    )(_sc_body)
    return fn(emb2, idx_i, idx_j, params)


@jax.jit
def kernel(idx_i, idx_j, precomputed_features, feature_weights, intercept):
    featsT = jnp.transpose(
        precomputed_features, (0, 1, 3, 2)).reshape(K6, N_AUTHORS)
    emb2 = _tc_stage(featsT, feature_weights.astype(jnp.float32))
    params = jnp.concatenate([
        intercept.reshape(1).astype(jnp.float32),
        jnp.zeros((15,), jnp.float32),
    ])
    return _sc_stage(emb2, idx_i, idx_j, params)
